# Initial kernel scaffold; baseline (speedup 1.0000x reference)
import jax
import jax.numpy as jnp
from jax.experimental import pallas as pl


def kernel(x, edge_index, W, bias):
    raise NotImplementedError("write your pallas kernel here")


# speedup vs baseline: 3.6385x; 3.6385x over previous
import functools

import jax
import jax.numpy as jnp
from jax import lax
from jax.experimental import pallas as pl
from jax.experimental.pallas import tpu as pltpu
from jax.experimental.pallas import tpu_sc as plsc

_GROUPS = 4
_LANES = 16
_NC, _NS = 2, 16
_NW = _NC * _NS


def _sc_maxrel(xT, idx_j, idx_i, M, C, K):
    E = M * K
    EC = 128
    NPC = EC // K
    NCH = E // EC
    T = -(-NCH // _NW)

    mesh = plsc.VectorSubcoreMesh(core_axis_name="c", subcore_axis_name="s")

    @functools.partial(
        pl.kernel,
        mesh=mesh,
        out_type=jax.ShapeDtypeStruct((M, C), jnp.float32),
        scratch_types=[
            pltpu.VMEM((EC,), jnp.int32),
            pltpu.VMEM((EC,), jnp.int32),
            pltpu.VMEM((EC, C), jnp.float32),
            pltpu.VMEM((EC, C), jnp.float32),
            pltpu.VMEM((NPC, C), jnp.float32),
            pltpu.SemaphoreType.DMA,
            pltpu.SemaphoreType.DMA,
        ],
    )
    def sc_kernel(xT_hbm, ij_hbm, ii_hbm, out_hbm,
                  ij_v, ii_v, rj_v, ri_v, o_v, semj, semi):
        wid = lax.axis_index("s") * _NC + lax.axis_index("c")

        def body(t, carry):
            ch = wid + t * _NW

            @pl.when(ch < NCH)
            def _():
                e0 = pl.multiple_of(ch * EC, EC)
                pltpu.sync_copy(ij_hbm.at[pl.ds(e0, EC)], ij_v)
                pltpu.sync_copy(ii_hbm.at[pl.ds(e0, EC)], ii_v)
                cj = pltpu.async_copy(xT_hbm.at[ij_v], rj_v, semj)
                ci = pltpu.async_copy(xT_hbm.at[ii_v], ri_v, semi)
                cj.wait()
                ci.wait()

                def node(n, c2):
                    for cc in range(C // _LANES):
                        sl = pl.ds(cc * _LANES, _LANES)
                        acc = rj_v[n * K, sl] - ri_v[n * K, sl]
                        for kk in range(1, K):
                            acc = jnp.maximum(
                                acc, rj_v[n * K + kk, sl] - ri_v[n * K + kk, sl])
                        o_v[n, sl] = acc
                    return c2

                lax.fori_loop(0, NPC, node, 0)
                r0 = pl.multiple_of(ch * NPC, NPC)
                pltpu.sync_copy(o_v, out_hbm.at[pl.ds(r0, NPC)])

            return carry

        lax.fori_loop(0, T, body, 0)

    return sc_kernel(xT, idx_j, idx_i)


def _tc_body(wx_ref, wj_ref, b_ref, x_ref, mr_ref, o_ref):
    xb = x_ref[0]
    mr = mr_ref[0]
    acc = jnp.dot(wx_ref[...], xb, preferred_element_type=jnp.float32)
    acc = acc + lax.dot_general(
        wj_ref[...], mr, (((1,), (1,)), ((), ())),
        preferred_element_type=jnp.float32)
    o_ref[0] = jnp.maximum(acc + b_ref[...], 0.0)


def _tc_conv(x3, mr3, Wx, Wj, bias):
    B, C, N = x3.shape
    COUT = Wx.shape[0]
    grid = (B,)
    return pl.pallas_call(
        _tc_body,
        grid=grid,
        in_specs=[
            pl.BlockSpec((COUT, C), lambda b: (0, 0)),
            pl.BlockSpec((COUT, C), lambda b: (0, 0)),
            pl.BlockSpec((COUT, 1), lambda b: (0, 0)),
            pl.BlockSpec((1, C, N), lambda b: (b, 0, 0)),
            pl.BlockSpec((1, N, C), lambda b: (b, 0, 0)),
        ],
        out_specs=pl.BlockSpec((1, COUT, N), lambda b: (b, 0, 0)),
        out_shape=jax.ShapeDtypeStruct((B, COUT, N), jnp.float32),
    )(Wx, Wj, bias.reshape(COUT, 1), x3, mr3)


def _block_diag(blocks):
    G, R, S = blocks.shape
    out = jnp.zeros((G * R, G * S), blocks.dtype)
    for g in range(G):
        out = out.at[g * R:(g + 1) * R, g * S:(g + 1) * S].set(blocks[g])
    return out


def kernel(x, edge_index, W, bias):
    B, C, N, _ = x.shape
    K = edge_index.shape[-1]
    COUT = W.shape[0]

    x3 = x[..., 0]
    xT = jnp.transpose(x3, (0, 2, 1)).reshape(B * N, C)
    ei = edge_index.astype(jnp.int32)
    base = (jnp.arange(B, dtype=jnp.int32) * N)[:, None, None]
    idx_j = (ei[0] + base).reshape(B * N * K)
    idx_i = (ei[1] + base).reshape(B * N * K)

    mr = _sc_maxrel(xT, idx_j, idx_i, B * N, C, K)

    Wg = W[:, :, 0, 0].reshape(_GROUPS, COUT // _GROUPS, (2 * C) // _GROUPS)
    Wx = _block_diag(Wg[:, :, 0::2])
    Wj = _block_diag(Wg[:, :, 1::2])

    out = _tc_conv(x3, mr.reshape(B, N, C), Wx, Wj, bias)
    return out[..., None]
